# Initial kernel scaffold; baseline (speedup 1.0000x reference)
import jax
import jax.numpy as jnp
from jax.experimental import pallas as pl


def kernel(users, items, user_table, item_table, rows, cols, vals):
    raise NotImplementedError("write your pallas kernel here")


# speedup vs baseline: 3.4952x; 3.4952x over previous
import functools

import jax
import jax.numpy as jnp
from jax import lax
from jax.experimental import pallas as pl
from jax.experimental.pallas import tpu as pltpu
from jax.experimental.pallas import tpu_sc as plsc

NU = 5000
NI = 5000
EMB = 256
NL = 5
NE = 80000
NB = 4096
TPS = 16
NSC = 2
PSIDE = 5120
NP = 2 * PSIDE
RPT = PSIDE // TPS
CH = 128
NCH = 40
EPT = NCH * CH
EC = 128
WB = 64
VL = 16
BPW = NB // (NSC * TPS)

_mesh = plsc.VectorSubcoreMesh(core_axis_name="c", subcore_axis_name="s")
_f32 = jnp.float32


def _deg_body(rowsq, deg_out, deg_sh, idxv, onesv, stg):
  cid = lax.axis_index("c")
  sid = lax.axis_index("s")

  @pl.loop(0, RPT // VL)
  def _(i):
    stg[pl.ds(i * VL, VL)] = jnp.zeros((VL,), _f32)

  pltpu.sync_copy(stg, deg_sh.at[pl.ds(sid * RPT, RPT)])

  @pl.loop(0, CH // VL)
  def _(i):
    onesv[pl.ds(i * VL, VL)] = jnp.ones((VL,), _f32)

  pltpu.sync_copy(rowsq.at[cid, sid], idxv)
  plsc.subcore_barrier()

  @pl.loop(0, NCH)
  def _(j):
    pltpu.sync_copy(onesv, deg_sh.at[idxv.at[j]], add=True)

  plsc.subcore_barrier()
  pltpu.sync_copy(deg_sh.at[pl.ds(sid * RPT, RPT)], stg)
  pltpu.sync_copy(stg, deg_out.at[pl.ds(cid * PSIDE + sid * RPT, RPT)])


_deg_call = pl.kernel(
    _deg_body,
    out_type=jax.ShapeDtypeStruct((NP,), _f32),
    mesh=_mesh,
    scratch_types=[
        pltpu.VMEM_SHARED((PSIDE,), _f32),
        pltpu.VMEM((NCH, CH), jnp.int32),
        pltpu.VMEM((CH,), _f32),
        pltpu.VMEM((RPT,), _f32),
    ],
)


def _prep_body(deg_ref, emb_ref, dinv_ref, dinv2_ref, z0_ref, z1_ref):
  d = deg_ref[...]
  dinv = jnp.where(d > 0.0, lax.rsqrt(jnp.maximum(d, 1e-30)), 0.0)
  dinv_ref[...] = dinv
  dinv2_ref[...] = dinv * dinv
  z0_ref[...] = emb_ref[:, :EC] * dinv
  z1_ref[...] = emb_ref[:, EC:] * dinv


_prep_call = pl.pallas_call(
    _prep_body,
    out_shape=(
        jax.ShapeDtypeStruct((NP, 1), _f32),
        jax.ShapeDtypeStruct((NP, 1), _f32),
        jax.ShapeDtypeStruct((NP, EC), _f32),
        jax.ShapeDtypeStruct((NP, EC), _f32),
    ),
)


def _layer_body(z0, z1, acc, rowsq, colsq, dinv, dinv2,
                z0out, z1out, accout,
                acc_sh, gbuf0, gbuf1, rv, cv, sbuf, abuf, dv, dv2,
                sem0, sem1):
  cid = lax.axis_index("c")
  sid = lax.axis_index("s")
  gbase = cid * PSIDE + sid * RPT
  lbase = sid * RPT

  pltpu.sync_copy(rowsq.at[cid, sid], rv)
  pltpu.sync_copy(colsq.at[cid, sid], cv)
  pltpu.sync_copy(dinv.at[pl.ds(gbase, RPT)], dv)
  pltpu.sync_copy(dinv2.at[pl.ds(gbase, RPT)], dv2)

  for h, zin, zo in ((0, z0, z0out), (1, z1, z1out)):
    @pl.loop(0, WB)
    def _(r):
      for v in range(EC // VL):
        sbuf[r, pl.ds(v * VL, VL)] = jnp.zeros((VL,), _f32)

    for k in range(RPT // WB):
      pltpu.sync_copy(sbuf, acc_sh.at[pl.ds(lbase + k * WB, WB)])
    plsc.subcore_barrier()

    @pl.loop(0, NCH // 2)
    def _(jo):
      j0 = jo * 2
      d0 = pltpu.async_copy(zin.at[cv.at[j0]], gbuf0, sem0)
      d1 = pltpu.async_copy(zin.at[cv.at[j0 + 1]], gbuf1, sem1)
      d0.wait()
      pltpu.sync_copy(gbuf0, acc_sh.at[rv.at[j0]], add=True)
      d1.wait()
      pltpu.sync_copy(gbuf1, acc_sh.at[rv.at[j0 + 1]], add=True)

    plsc.subcore_barrier()

    for k in range(RPT // WB):
      pltpu.sync_copy(acc_sh.at[pl.ds(lbase + k * WB, WB)], sbuf)
      pltpu.sync_copy(
          acc.at[pl.ds(gbase + k * WB, WB), pl.ds(h * EC, EC)], abuf)

      @pl.loop(0, WB // VL)
      def _(rg):
        dvv = dv[pl.ds(k * WB + rg * VL, VL)]
        dv2v = dv2[pl.ds(k * WB + rg * VL, VL)]
        for ri in range(VL):
          r = rg * VL + ri
          di = dvv[ri]
          di2 = dv2v[ri]
          for v in range(EC // VL):
            s = sbuf[r, pl.ds(v * VL, VL)]
            a = abuf[r, pl.ds(v * VL, VL)]
            sbuf[r, pl.ds(v * VL, VL)] = di2 * s
            abuf[r, pl.ds(v * VL, VL)] = a + di * s

      pltpu.sync_copy(sbuf, zo.at[pl.ds(gbase + k * WB, WB)])
      pltpu.sync_copy(
          abuf, accout.at[pl.ds(gbase + k * WB, WB), pl.ds(h * EC, EC)])


_layer_call = pl.kernel(
    _layer_body,
    out_type=(
        jax.ShapeDtypeStruct((NP, EC), _f32),
        jax.ShapeDtypeStruct((NP, EC), _f32),
        jax.ShapeDtypeStruct((NP, EMB), _f32),
    ),
    mesh=_mesh,
    scratch_types=[
        pltpu.VMEM_SHARED((PSIDE, EC), _f32),
        pltpu.VMEM((CH, EC), _f32),
        pltpu.VMEM((CH, EC), _f32),
        pltpu.VMEM((NCH, CH), jnp.int32),
        pltpu.VMEM((NCH, CH), jnp.int32),
        pltpu.VMEM((WB, EC), _f32),
        pltpu.VMEM((WB, EC), _f32),
        pltpu.VMEM((RPT,), _f32),
        pltpu.VMEM((RPT,), _f32),
        pltpu.SemaphoreType.DMA,
        pltpu.SemaphoreType.DMA,
    ],
)


def _out_body(acc, uidx, iidx, uout, iout, idxv, buf, sem):
  cid = lax.axis_index("c")
  sid = lax.axis_index("s")
  base = (cid * TPS + sid) * BPW

  for src, dst in ((uidx, uout), (iidx, iout)):
    pltpu.sync_copy(src.at[pl.ds(base, BPW)], idxv)
    pltpu.async_copy(acc.at[idxv], buf, sem).wait()

    @pl.loop(0, BPW)
    def _(r):
      for v in range(EMB // VL):
        buf[r, pl.ds(v * VL, VL)] = buf[r, pl.ds(v * VL, VL)] * (1.0 / 6.0)

    pltpu.sync_copy(buf, dst.at[pl.ds(base, BPW)])


_out_call = pl.kernel(
    _out_body,
    out_type=(
        jax.ShapeDtypeStruct((NB, EMB), _f32),
        jax.ShapeDtypeStruct((NB, EMB), _f32),
    ),
    mesh=_mesh,
    scratch_types=[
        pltpu.VMEM((BPW,), jnp.int32),
        pltpu.VMEM((BPW, EMB), _f32),
        pltpu.SemaphoreType.DMA,
    ],
)


def _pack_side(r_local, c_padded):
  npad = TPS * EPT - NE
  r = jnp.concatenate([r_local, jnp.full((npad,), PSIDE - 1, jnp.int32)])
  c = jnp.concatenate([c_padded, jnp.full((npad,), NP - 1, jnp.int32)])
  return r.reshape(TPS, NCH, CH), c.reshape(TPS, NCH, CH)


def kernel(users, items, user_table, item_table, rows, cols, vals):
  users = users.astype(jnp.int32)
  items = items.astype(jnp.int32)
  rows = rows.astype(jnp.int32)
  cols = cols.astype(jnp.int32)

  emb = jnp.concatenate([
      jnp.pad(user_table.astype(_f32), ((0, PSIDE - NU), (0, 0))),
      jnp.pad(item_table.astype(_f32), ((0, PSIDE - NI), (0, 0))),
  ], axis=0)

  r0, c0 = _pack_side(rows[:NE], cols[:NE] + (PSIDE - NU))
  r1, c1 = _pack_side(rows[NE:] - NU, cols[NE:])
  rowsq = jnp.stack([r0, r1])
  colsq = jnp.stack([c0, c1])

  deg = _deg_call(rowsq).reshape(NP, 1)
  dinv, dinv2, z0h, z1h = _prep_call(deg, emb)
  dinv = dinv.reshape(NP)
  dinv2 = dinv2.reshape(NP)

  acc = emb
  for _ in range(NL):
    z0h, z1h, acc = _layer_call(z0h, z1h, acc, rowsq, colsq, dinv, dinv2)

  return _out_call(acc, users, items + PSIDE)


# speedup vs baseline: 3.7282x; 1.0667x over previous
import functools

import jax
import jax.numpy as jnp
from jax import lax
from jax.experimental import pallas as pl
from jax.experimental.pallas import tpu as pltpu
from jax.experimental.pallas import tpu_sc as plsc

NU = 5000
NI = 5000
EMB = 256
NL = 5
NE = 80000
NB = 4096
TPS = 16
NSC = 2
PSIDE = 5120
NP = 2 * PSIDE
RPT = PSIDE // TPS
CH = 128
NCH = 40
EPT = NCH * CH
EC = 128
NBUF = 3
LAG = 2
WB = 64
VL = 16
BPW = NB // (NSC * TPS)

_mesh = plsc.VectorSubcoreMesh(core_axis_name="c", subcore_axis_name="s")
_f32 = jnp.float32


def _deg_body(rowsq, deg_out, deg_sh, idxv, onesv, stg):
  cid = lax.axis_index("c")
  sid = lax.axis_index("s")

  @pl.loop(0, RPT // VL)
  def _(i):
    stg[pl.ds(i * VL, VL)] = jnp.zeros((VL,), _f32)

  pltpu.sync_copy(stg, deg_sh.at[pl.ds(sid * RPT, RPT)])

  @pl.loop(0, CH // VL)
  def _(i):
    onesv[pl.ds(i * VL, VL)] = jnp.ones((VL,), _f32)

  pltpu.sync_copy(rowsq.at[cid, sid], idxv)
  plsc.subcore_barrier()

  @pl.loop(0, NCH)
  def _(j):
    pltpu.sync_copy(onesv, deg_sh.at[idxv.at[j]], add=True)

  plsc.subcore_barrier()
  pltpu.sync_copy(deg_sh.at[pl.ds(sid * RPT, RPT)], stg)
  pltpu.sync_copy(stg, deg_out.at[pl.ds(cid * PSIDE + sid * RPT, RPT)])


_deg_call = pl.kernel(
    _deg_body,
    out_type=jax.ShapeDtypeStruct((NP,), _f32),
    mesh=_mesh,
    scratch_types=[
        pltpu.VMEM_SHARED((PSIDE,), _f32),
        pltpu.VMEM((NCH, CH), jnp.int32),
        pltpu.VMEM((CH,), _f32),
        pltpu.VMEM((RPT,), _f32),
    ],
)


def _prep_body(deg_ref, emb_ref, dinv_ref, dinv2_ref, z0_ref, z1_ref):
  d = deg_ref[...]
  dinv = jnp.where(d > 0.0, lax.rsqrt(jnp.maximum(d, 1e-30)), 0.0)
  dinv_ref[...] = dinv
  dinv2_ref[...] = dinv * dinv
  z0_ref[...] = emb_ref[:, :EC] * dinv
  z1_ref[...] = emb_ref[:, EC:] * dinv


_prep_call = pl.pallas_call(
    _prep_body,
    out_shape=(
        jax.ShapeDtypeStruct((NP, 1), _f32),
        jax.ShapeDtypeStruct((NP, 1), _f32),
        jax.ShapeDtypeStruct((NP, EC), _f32),
        jax.ShapeDtypeStruct((NP, EC), _f32),
    ),
)


def _layer_body(z0, z1, acc, rowsq, colsq, dinv, dinv2,
                z0out, z1out, accout,
                acc_sh, g0, g1, g2, rv, cv, sbuf, abuf, dv, dv2,
                gs0, gs1, gs2, ss0, ss1, ss2):
  gbufs = (g0, g1, g2)
  gsems = (gs0, gs1, gs2)
  ssems = (ss0, ss1, ss2)
  cid = lax.axis_index("c")
  sid = lax.axis_index("s")
  gbase = cid * PSIDE + sid * RPT
  lbase = sid * RPT

  pltpu.sync_copy(rowsq.at[cid, sid], rv)
  pltpu.sync_copy(colsq.at[cid, sid], cv)
  pltpu.sync_copy(dinv.at[pl.ds(gbase, RPT)], dv)
  pltpu.sync_copy(dinv2.at[pl.ds(gbase, RPT)], dv2)

  for h, zin, zo in ((0, z0, z0out), (1, z1, z1out)):
    @pl.loop(0, WB)
    def _(r):
      for v in range(EC // VL):
        sbuf[r, pl.ds(v * VL, VL)] = jnp.zeros((VL,), _f32)

    for k in range(RPT // WB):
      pltpu.sync_copy(sbuf, acc_sh.at[pl.ds(lbase + k * WB, WB)])
    plsc.subcore_barrier()

    gh = [None] * NBUF
    sh = [None] * NBUF
    for j in range(NCH + LAG):
      if j < NCH:
        b = j % NBUF
        if sh[b] is not None:
          sh[b].wait()
          sh[b] = None
        gh[b] = pltpu.async_copy(zin.at[cv.at[j]], gbufs[b], gsems[b])
      if j >= LAG:
        jj = j - LAG
        b = jj % NBUF
        gh[b].wait()
        gh[b] = None
        sh[b] = pltpu.async_copy(gbufs[b], acc_sh.at[rv.at[jj]], ssems[b],
                                 add=True)
    for b in range(NBUF):
      if sh[b] is not None:
        sh[b].wait()

    plsc.subcore_barrier()

    for k in range(RPT // WB):
      pltpu.sync_copy(acc_sh.at[pl.ds(lbase + k * WB, WB)], sbuf)
      pltpu.sync_copy(
          acc.at[pl.ds(gbase + k * WB, WB), pl.ds(h * EC, EC)], abuf)

      @pl.loop(0, WB // VL)
      def _(rg):
        dvv = dv[pl.ds(k * WB + rg * VL, VL)]
        dv2v = dv2[pl.ds(k * WB + rg * VL, VL)]
        for ri in range(VL):
          r = rg * VL + ri
          di = dvv[ri]
          di2 = dv2v[ri]
          for v in range(EC // VL):
            s = sbuf[r, pl.ds(v * VL, VL)]
            a = abuf[r, pl.ds(v * VL, VL)]
            sbuf[r, pl.ds(v * VL, VL)] = di2 * s
            abuf[r, pl.ds(v * VL, VL)] = a + di * s

      pltpu.sync_copy(sbuf, zo.at[pl.ds(gbase + k * WB, WB)])
      pltpu.sync_copy(
          abuf, accout.at[pl.ds(gbase + k * WB, WB), pl.ds(h * EC, EC)])


_layer_call = pl.kernel(
    _layer_body,
    out_type=(
        jax.ShapeDtypeStruct((NP, EC), _f32),
        jax.ShapeDtypeStruct((NP, EC), _f32),
        jax.ShapeDtypeStruct((NP, EMB), _f32),
    ),
    mesh=_mesh,
    scratch_types=[
        pltpu.VMEM_SHARED((PSIDE, EC), _f32),
        pltpu.VMEM((CH, EC), _f32),
        pltpu.VMEM((CH, EC), _f32),
        pltpu.VMEM((CH, EC), _f32),
        pltpu.VMEM((NCH, CH), jnp.int32),
        pltpu.VMEM((NCH, CH), jnp.int32),
        pltpu.VMEM((WB, EC), _f32),
        pltpu.VMEM((WB, EC), _f32),
        pltpu.VMEM((RPT,), _f32),
        pltpu.VMEM((RPT,), _f32),
    ] + [pltpu.SemaphoreType.DMA] * (2 * NBUF),
)


def _out_body(acc, uidx, iidx, uout, iout, idxv, buf, sem):
  cid = lax.axis_index("c")
  sid = lax.axis_index("s")
  base = (cid * TPS + sid) * BPW

  for src, dst in ((uidx, uout), (iidx, iout)):
    pltpu.sync_copy(src.at[pl.ds(base, BPW)], idxv)
    pltpu.async_copy(acc.at[idxv], buf, sem).wait()

    @pl.loop(0, BPW)
    def _(r):
      for v in range(EMB // VL):
        buf[r, pl.ds(v * VL, VL)] = buf[r, pl.ds(v * VL, VL)] * (1.0 / 6.0)

    pltpu.sync_copy(buf, dst.at[pl.ds(base, BPW)])


_out_call = pl.kernel(
    _out_body,
    out_type=(
        jax.ShapeDtypeStruct((NB, EMB), _f32),
        jax.ShapeDtypeStruct((NB, EMB), _f32),
    ),
    mesh=_mesh,
    scratch_types=[
        pltpu.VMEM((BPW,), jnp.int32),
        pltpu.VMEM((BPW, EMB), _f32),
        pltpu.SemaphoreType.DMA,
    ],
)


def _pack_side(r_local, c_padded):
  npad = TPS * EPT - NE
  r = jnp.concatenate([r_local, jnp.full((npad,), PSIDE - 1, jnp.int32)])
  c = jnp.concatenate([c_padded, jnp.full((npad,), NP - 1, jnp.int32)])
  return r.reshape(TPS, NCH, CH), c.reshape(TPS, NCH, CH)


def kernel(users, items, user_table, item_table, rows, cols, vals):
  users = users.astype(jnp.int32)
  items = items.astype(jnp.int32)
  rows = rows.astype(jnp.int32)
  cols = cols.astype(jnp.int32)

  emb = jnp.concatenate([
      jnp.pad(user_table.astype(_f32), ((0, PSIDE - NU), (0, 0))),
      jnp.pad(item_table.astype(_f32), ((0, PSIDE - NI), (0, 0))),
  ], axis=0)

  r0, c0 = _pack_side(rows[:NE], cols[:NE] + (PSIDE - NU))
  r1, c1 = _pack_side(rows[NE:] - NU, cols[NE:])
  rowsq = jnp.stack([r0, r1])
  colsq = jnp.stack([c0, c1])

  deg = _deg_call(rowsq).reshape(NP, 1)
  dinv, dinv2, z0h, z1h = _prep_call(deg, emb)
  dinv = dinv.reshape(NP)
  dinv2 = dinv2.reshape(NP)

  acc = emb
  for _ in range(NL):
    z0h, z1h, acc = _layer_call(z0h, z1h, acc, rowsq, colsq, dinv, dinv2)

  return _out_call(acc, users, items + PSIDE)


# speedup vs baseline: 7.5242x; 2.0182x over previous
import functools

import jax
import jax.numpy as jnp
from jax import lax
from jax.experimental import pallas as pl
from jax.experimental.pallas import tpu as pltpu
from jax.experimental.pallas import tpu_sc as plsc

NU = 5000
NI = 5000
EMB = 256
NL = 5
NE = 80000
NB = 4096
TPS = 16
NSC = 2
PSIDE = 5120
NP = 2 * PSIDE
RPT = PSIDE // TPS
CH = 64
NCH = 80
EPT = NCH * CH
EC = 128
NBUF = 2
LAG = 1
WB = 32
VL = 16
BPW = NB // (NSC * TPS)

_mesh = plsc.VectorSubcoreMesh(core_axis_name="c", subcore_axis_name="s")
_f32 = jnp.float32


def _deg_body(rowsq, deg_out, deg_sh, idxv, onesv, stg):
  cid = lax.axis_index("c")
  sid = lax.axis_index("s")

  @pl.loop(0, RPT // VL)
  def _(i):
    stg[pl.ds(i * VL, VL)] = jnp.zeros((VL,), _f32)

  pltpu.sync_copy(stg, deg_sh.at[pl.ds(sid * RPT, RPT)])

  @pl.loop(0, CH // VL)
  def _(i):
    onesv[pl.ds(i * VL, VL)] = jnp.ones((VL,), _f32)

  pltpu.sync_copy(rowsq.at[cid, sid], idxv)
  plsc.subcore_barrier()

  @pl.loop(0, NCH)
  def _(j):
    pltpu.sync_copy(onesv, deg_sh.at[idxv.at[j]], add=True)

  plsc.subcore_barrier()
  pltpu.sync_copy(deg_sh.at[pl.ds(sid * RPT, RPT)], stg)
  pltpu.sync_copy(stg, deg_out.at[pl.ds(cid * PSIDE + sid * RPT, RPT)])


_deg_call = pl.kernel(
    _deg_body,
    out_type=jax.ShapeDtypeStruct((NP,), _f32),
    mesh=_mesh,
    scratch_types=[
        pltpu.VMEM_SHARED((PSIDE,), _f32),
        pltpu.VMEM((NCH, CH), jnp.int32),
        pltpu.VMEM((CH,), _f32),
        pltpu.VMEM((RPT,), _f32),
    ],
)


def _prep_body(deg_ref, emb_ref, dinv_ref, dinv2_ref, z0_ref, z1_ref):
  d = deg_ref[...]
  dinv = jnp.where(d > 0.0, lax.rsqrt(jnp.maximum(d, 1e-30)), 0.0)
  dinv_ref[...] = dinv
  dinv2_ref[...] = dinv * dinv
  z0_ref[...] = emb_ref[:, :EC] * dinv
  z1_ref[...] = emb_ref[:, EC:] * dinv


_prep_call = pl.pallas_call(
    _prep_body,
    out_shape=(
        jax.ShapeDtypeStruct((NP, 1), _f32),
        jax.ShapeDtypeStruct((NP, 1), _f32),
        jax.ShapeDtypeStruct((NP, EC), _f32),
        jax.ShapeDtypeStruct((NP, EC), _f32),
    ),
)


def _layer_body(z0, z1, acc, rowsq, colsq, dinv, dinv2,
                z0out, z1out, accout,
                acc_sh, z_sh, g0, g1, rv, cv, sbuf, abuf, dv, dv2,
                gs0, gs1, ss0, ss1):
  gbufs = (g0, g1)
  gsems = (gs0, gs1)
  ssems = (ss0, ss1)
  cid = lax.axis_index("c")
  sid = lax.axis_index("s")
  gbase = cid * PSIDE + sid * RPT
  lbase = sid * RPT
  sbase = (1 - cid) * PSIDE + sid * RPT

  pltpu.sync_copy(rowsq.at[cid, sid], rv)
  pltpu.sync_copy(colsq.at[cid, sid], cv)
  pltpu.sync_copy(dinv.at[pl.ds(gbase, RPT)], dv)
  pltpu.sync_copy(dinv2.at[pl.ds(gbase, RPT)], dv2)

  for h, zin, zo in ((0, z0, z0out), (1, z1, z1out)):
    pltpu.sync_copy(zin.at[pl.ds(sbase, RPT)], z_sh.at[pl.ds(lbase, RPT)])

    @pl.loop(0, WB)
    def _(r):
      for v in range(EC // VL):
        sbuf[r, pl.ds(v * VL, VL)] = jnp.zeros((VL,), _f32)

    for k in range(RPT // WB):
      pltpu.sync_copy(sbuf, acc_sh.at[pl.ds(lbase + k * WB, WB)])
    plsc.subcore_barrier()

    gh = [None] * NBUF
    sh = [None] * NBUF
    for j in range(NCH + LAG):
      if j < NCH:
        b = j % NBUF
        if sh[b] is not None:
          sh[b].wait()
          sh[b] = None
        gh[b] = pltpu.async_copy(z_sh.at[cv.at[j]], gbufs[b], gsems[b])
      if j >= LAG:
        jj = j - LAG
        b = jj % NBUF
        gh[b].wait()
        gh[b] = None
        sh[b] = pltpu.async_copy(gbufs[b], acc_sh.at[rv.at[jj]], ssems[b],
                                 add=True)
    for b in range(NBUF):
      if sh[b] is not None:
        sh[b].wait()

    plsc.subcore_barrier()

    for k in range(RPT // WB):
      pltpu.sync_copy(acc_sh.at[pl.ds(lbase + k * WB, WB)], sbuf)
      pltpu.sync_copy(
          acc.at[pl.ds(gbase + k * WB, WB), pl.ds(h * EC, EC)], abuf)

      @pl.loop(0, WB // VL)
      def _(rg):
        dvv = dv[pl.ds(k * WB + rg * VL, VL)]
        dv2v = dv2[pl.ds(k * WB + rg * VL, VL)]
        for ri in range(VL):
          r = rg * VL + ri
          di = dvv[ri]
          di2 = dv2v[ri]
          for v in range(EC // VL):
            s = sbuf[r, pl.ds(v * VL, VL)]
            a = abuf[r, pl.ds(v * VL, VL)]
            sbuf[r, pl.ds(v * VL, VL)] = di2 * s
            abuf[r, pl.ds(v * VL, VL)] = a + di * s

      pltpu.sync_copy(sbuf, zo.at[pl.ds(gbase + k * WB, WB)])
      pltpu.sync_copy(
          abuf, accout.at[pl.ds(gbase + k * WB, WB), pl.ds(h * EC, EC)])


_layer_call = pl.kernel(
    _layer_body,
    out_type=(
        jax.ShapeDtypeStruct((NP, EC), _f32),
        jax.ShapeDtypeStruct((NP, EC), _f32),
        jax.ShapeDtypeStruct((NP, EMB), _f32),
    ),
    mesh=_mesh,
    scratch_types=[
        pltpu.VMEM_SHARED((PSIDE, EC), _f32),
        pltpu.VMEM_SHARED((PSIDE, EC), _f32),
        pltpu.VMEM((CH, EC), _f32),
        pltpu.VMEM((CH, EC), _f32),
        pltpu.VMEM((NCH, CH), jnp.int32),
        pltpu.VMEM((NCH, CH), jnp.int32),
        pltpu.VMEM((WB, EC), _f32),
        pltpu.VMEM((WB, EC), _f32),
        pltpu.VMEM((RPT,), _f32),
        pltpu.VMEM((RPT,), _f32),
    ] + [pltpu.SemaphoreType.DMA] * (2 * NBUF),
)


def _out_body(acc, uidx, iidx, uout, iout, idxv, buf, sem):
  cid = lax.axis_index("c")
  sid = lax.axis_index("s")
  base = (cid * TPS + sid) * BPW

  for src, dst in ((uidx, uout), (iidx, iout)):
    pltpu.sync_copy(src.at[pl.ds(base, BPW)], idxv)
    pltpu.async_copy(acc.at[idxv], buf, sem).wait()

    @pl.loop(0, BPW)
    def _(r):
      for v in range(EMB // VL):
        buf[r, pl.ds(v * VL, VL)] = buf[r, pl.ds(v * VL, VL)] * (1.0 / 6.0)

    pltpu.sync_copy(buf, dst.at[pl.ds(base, BPW)])


_out_call = pl.kernel(
    _out_body,
    out_type=(
        jax.ShapeDtypeStruct((NB, EMB), _f32),
        jax.ShapeDtypeStruct((NB, EMB), _f32),
    ),
    mesh=_mesh,
    scratch_types=[
        pltpu.VMEM((BPW,), jnp.int32),
        pltpu.VMEM((BPW, EMB), _f32),
        pltpu.SemaphoreType.DMA,
    ],
)


def _pack_side(r_local, c_local):
  npad = TPS * EPT - NE
  r = jnp.concatenate([r_local, jnp.full((npad,), PSIDE - 1, jnp.int32)])
  c = jnp.concatenate([c_local, jnp.full((npad,), PSIDE - 1, jnp.int32)])
  return r.reshape(TPS, NCH, CH), c.reshape(TPS, NCH, CH)


def kernel(users, items, user_table, item_table, rows, cols, vals):
  users = users.astype(jnp.int32)
  items = items.astype(jnp.int32)
  rows = rows.astype(jnp.int32)
  cols = cols.astype(jnp.int32)

  emb = jnp.concatenate([
      jnp.pad(user_table.astype(_f32), ((0, PSIDE - NU), (0, 0))),
      jnp.pad(item_table.astype(_f32), ((0, PSIDE - NI), (0, 0))),
  ], axis=0)

  r0, c0 = _pack_side(rows[:NE], cols[:NE] - NU)
  r1, c1 = _pack_side(rows[NE:] - NU, cols[NE:])
  rowsq = jnp.stack([r0, r1])
  colsq = jnp.stack([c0, c1])

  deg = _deg_call(rowsq).reshape(NP, 1)
  dinv, dinv2, z0h, z1h = _prep_call(deg, emb)
  dinv = dinv.reshape(NP)
  dinv2 = dinv2.reshape(NP)

  acc = emb
  for _ in range(NL):
    z0h, z1h, acc = _layer_call(z0h, z1h, acc, rowsq, colsq, dinv, dinv2)

  return _out_call(acc, users, items + PSIDE)


# speedup vs baseline: 8.1012x; 1.0767x over previous
import functools

import jax
import jax.numpy as jnp
from jax import lax
from jax.experimental import pallas as pl
from jax.experimental.pallas import tpu as pltpu
from jax.experimental.pallas import tpu_sc as plsc

NU = 5000
NI = 5000
EMB = 256
NL = 5
NE = 80000
NB = 4096
TPS = 16
NSC = 2
PSIDE = 5120
NP = 2 * PSIDE
RPT = PSIDE // TPS
CH = 64
NCH = 80
EPT = NCH * CH
EC = 128
NBUF = 3
LAG = NBUF - 1
WB = CH
VL = 16
BPW = NB // (NSC * TPS)

_mesh = plsc.VectorSubcoreMesh(core_axis_name="c", subcore_axis_name="s")
_f32 = jnp.float32


def _deg_body(rowsq, deg_out, deg_sh, idxv, onesv, stg):
  cid = lax.axis_index("c")
  sid = lax.axis_index("s")

  @pl.loop(0, RPT // VL)
  def _(i):
    stg[pl.ds(i * VL, VL)] = jnp.zeros((VL,), _f32)

  pltpu.sync_copy(stg, deg_sh.at[pl.ds(sid * RPT, RPT)])

  @pl.loop(0, CH // VL)
  def _(i):
    onesv[pl.ds(i * VL, VL)] = jnp.ones((VL,), _f32)

  pltpu.sync_copy(rowsq.at[cid, sid], idxv)
  plsc.subcore_barrier()

  @pl.loop(0, NCH)
  def _(j):
    pltpu.sync_copy(onesv, deg_sh.at[idxv.at[j]], add=True)

  plsc.subcore_barrier()
  pltpu.sync_copy(deg_sh.at[pl.ds(sid * RPT, RPT)], stg)
  pltpu.sync_copy(stg, deg_out.at[pl.ds(cid * PSIDE + sid * RPT, RPT)])


_deg_call = pl.kernel(
    _deg_body,
    out_type=jax.ShapeDtypeStruct((NP,), _f32),
    mesh=_mesh,
    scratch_types=[
        pltpu.VMEM_SHARED((PSIDE,), _f32),
        pltpu.VMEM((NCH, CH), jnp.int32),
        pltpu.VMEM((CH,), _f32),
        pltpu.VMEM((RPT,), _f32),
    ],
)


def _prep_body(deg_ref, emb_ref, dinv_ref, dinv2_ref, z0_ref, z1_ref):
  d = deg_ref[...]
  dinv = jnp.where(d > 0.0, lax.rsqrt(jnp.maximum(d, 1e-30)), 0.0)
  dinv_ref[...] = dinv
  dinv2_ref[...] = dinv * dinv
  z0_ref[...] = emb_ref[:, :EC] * dinv
  z1_ref[...] = emb_ref[:, EC:] * dinv


_prep_call = pl.pallas_call(
    _prep_body,
    out_shape=(
        jax.ShapeDtypeStruct((NP, 1), _f32),
        jax.ShapeDtypeStruct((NP, 1), _f32),
        jax.ShapeDtypeStruct((NP, EC), _f32),
        jax.ShapeDtypeStruct((NP, EC), _f32),
    ),
)


def _layer_body(z0, z1, acc, rowsq, colsq, dinv, dinv2,
                z0out, z1out, accout,
                acc_sh, z_sh, rv, cv, dv, dv2, *gbs):
  gbufs = gbs[:NBUF]
  gsems = gbs[NBUF:2 * NBUF]
  ssems = gbs[2 * NBUF:]
  sbuf = gbufs[0]
  abuf = gbufs[1]
  cid = lax.axis_index("c")
  sid = lax.axis_index("s")
  gbase = cid * PSIDE + sid * RPT
  lbase = sid * RPT
  sbase = (1 - cid) * PSIDE + sid * RPT

  pltpu.sync_copy(rowsq.at[cid, sid], rv)
  pltpu.sync_copy(colsq.at[cid, sid], cv)
  pltpu.sync_copy(dinv.at[pl.ds(gbase, RPT)], dv)
  pltpu.sync_copy(dinv2.at[pl.ds(gbase, RPT)], dv2)

  for h, zin, zo in ((0, z0, z0out), (1, z1, z1out)):
    pltpu.sync_copy(zin.at[pl.ds(sbase, RPT)], z_sh.at[pl.ds(lbase, RPT)])

    @pl.loop(0, WB)
    def _(r):
      for v in range(EC // VL):
        sbuf[r, pl.ds(v * VL, VL)] = jnp.zeros((VL,), _f32)

    for k in range(RPT // WB):
      pltpu.sync_copy(sbuf, acc_sh.at[pl.ds(lbase + k * WB, WB)])
    plsc.subcore_barrier()

    gh = [None] * NBUF
    sh = [None] * NBUF
    for j in range(NCH + LAG):
      if j < NCH:
        b = j % NBUF
        if sh[b] is not None:
          sh[b].wait()
          sh[b] = None
        gh[b] = pltpu.async_copy(z_sh.at[cv.at[j]], gbufs[b], gsems[b])
      if j >= LAG:
        jj = j - LAG
        b = jj % NBUF
        gh[b].wait()
        gh[b] = None
        sh[b] = pltpu.async_copy(gbufs[b], acc_sh.at[rv.at[jj]], ssems[b],
                                 add=True)
    for b in range(NBUF):
      if sh[b] is not None:
        sh[b].wait()

    plsc.subcore_barrier()

    for k in range(RPT // WB):
      pltpu.sync_copy(acc_sh.at[pl.ds(lbase + k * WB, WB)], sbuf)
      pltpu.sync_copy(
          acc.at[pl.ds(gbase + k * WB, WB), pl.ds(h * EC, EC)], abuf)

      @pl.loop(0, WB // VL)
      def _(rg):
        dvv = dv[pl.ds(k * WB + rg * VL, VL)]
        dv2v = dv2[pl.ds(k * WB + rg * VL, VL)]
        for ri in range(VL):
          r = rg * VL + ri
          di = dvv[ri]
          di2 = dv2v[ri]
          for v in range(EC // VL):
            s = sbuf[r, pl.ds(v * VL, VL)]
            a = abuf[r, pl.ds(v * VL, VL)]
            sbuf[r, pl.ds(v * VL, VL)] = di2 * s
            abuf[r, pl.ds(v * VL, VL)] = a + di * s

      pltpu.sync_copy(sbuf, zo.at[pl.ds(gbase + k * WB, WB)])
      pltpu.sync_copy(
          abuf, accout.at[pl.ds(gbase + k * WB, WB), pl.ds(h * EC, EC)])


_layer_call = pl.kernel(
    _layer_body,
    out_type=(
        jax.ShapeDtypeStruct((NP, EC), _f32),
        jax.ShapeDtypeStruct((NP, EC), _f32),
        jax.ShapeDtypeStruct((NP, EMB), _f32),
    ),
    mesh=_mesh,
    scratch_types=[
        pltpu.VMEM_SHARED((PSIDE, EC), _f32),
        pltpu.VMEM_SHARED((PSIDE, EC), _f32),
        pltpu.VMEM((NCH, CH), jnp.int32),
        pltpu.VMEM((NCH, CH), jnp.int32),
        pltpu.VMEM((RPT,), _f32),
        pltpu.VMEM((RPT,), _f32),
    ] + [pltpu.VMEM((CH, EC), _f32)] * NBUF
      + [pltpu.SemaphoreType.DMA] * (2 * NBUF),
)


def _out_body(acc, uidx, iidx, uout, iout, idxv, buf, sem):
  cid = lax.axis_index("c")
  sid = lax.axis_index("s")
  base = (cid * TPS + sid) * BPW

  for src, dst in ((uidx, uout), (iidx, iout)):
    pltpu.sync_copy(src.at[pl.ds(base, BPW)], idxv)
    pltpu.async_copy(acc.at[idxv], buf, sem).wait()

    @pl.loop(0, BPW)
    def _(r):
      for v in range(EMB // VL):
        buf[r, pl.ds(v * VL, VL)] = buf[r, pl.ds(v * VL, VL)] * (1.0 / 6.0)

    pltpu.sync_copy(buf, dst.at[pl.ds(base, BPW)])


_out_call = pl.kernel(
    _out_body,
    out_type=(
        jax.ShapeDtypeStruct((NB, EMB), _f32),
        jax.ShapeDtypeStruct((NB, EMB), _f32),
    ),
    mesh=_mesh,
    scratch_types=[
        pltpu.VMEM((BPW,), jnp.int32),
        pltpu.VMEM((BPW, EMB), _f32),
        pltpu.SemaphoreType.DMA,
    ],
)


def _pack_side(r_local, c_local):
  npad = TPS * EPT - NE
  r = jnp.concatenate([r_local, jnp.full((npad,), PSIDE - 1, jnp.int32)])
  c = jnp.concatenate([c_local, jnp.full((npad,), PSIDE - 1, jnp.int32)])
  return r.reshape(TPS, NCH, CH), c.reshape(TPS, NCH, CH)


def kernel(users, items, user_table, item_table, rows, cols, vals):
  users = users.astype(jnp.int32)
  items = items.astype(jnp.int32)
  rows = rows.astype(jnp.int32)
  cols = cols.astype(jnp.int32)

  emb = jnp.concatenate([
      jnp.pad(user_table.astype(_f32), ((0, PSIDE - NU), (0, 0))),
      jnp.pad(item_table.astype(_f32), ((0, PSIDE - NI), (0, 0))),
  ], axis=0)

  r0, c0 = _pack_side(rows[:NE], cols[:NE] - NU)
  r1, c1 = _pack_side(rows[NE:] - NU, cols[NE:])
  rowsq = jnp.stack([r0, r1])
  colsq = jnp.stack([c0, c1])

  deg = _deg_call(rowsq).reshape(NP, 1)
  dinv, dinv2, z0h, z1h = _prep_call(deg, emb)
  dinv = dinv.reshape(NP)
  dinv2 = dinv2.reshape(NP)

  acc = emb
  for _ in range(NL):
    z0h, z1h, acc = _layer_call(z0h, z1h, acc, rowsq, colsq, dinv, dinv2)

  return _out_call(acc, users, items + PSIDE)


# speedup vs baseline: 8.8180x; 1.0885x over previous
import functools

import jax
import jax.numpy as jnp
from jax import lax
from jax.experimental import pallas as pl
from jax.experimental.pallas import tpu as pltpu
from jax.experimental.pallas import tpu_sc as plsc

NU = 5000
NI = 5000
EMB = 256
NL = 5
NE = 80000
NB = 4096
TPS = 16
NSC = 2
PSIDE = 5120
NP = 2 * PSIDE
RPT = PSIDE // TPS
CH = 64
NCH = 80
EPT = NCH * CH
EC = 128
NBUF = 3
LAG = NBUF - 1
WB = CH
VL = 16
BPW = NB // (NSC * TPS)

_mesh = plsc.VectorSubcoreMesh(core_axis_name="c", subcore_axis_name="s")
_f32 = jnp.float32


def _deg_body(rowsq, deg_out, deg_sh, idxv, onesv, stg):
  cid = lax.axis_index("c")
  sid = lax.axis_index("s")

  @pl.loop(0, RPT // VL)
  def _(i):
    stg[pl.ds(i * VL, VL)] = jnp.zeros((VL,), _f32)

  pltpu.sync_copy(stg, deg_sh.at[pl.ds(sid * RPT, RPT)])

  @pl.loop(0, CH // VL)
  def _(i):
    onesv[pl.ds(i * VL, VL)] = jnp.ones((VL,), _f32)

  pltpu.sync_copy(rowsq.at[cid, sid], idxv)
  plsc.subcore_barrier()

  @pl.loop(0, NCH)
  def _(j):
    pltpu.sync_copy(onesv, deg_sh.at[idxv.at[j]], add=True)

  plsc.subcore_barrier()
  pltpu.sync_copy(deg_sh.at[pl.ds(sid * RPT, RPT)], stg)
  pltpu.sync_copy(stg, deg_out.at[pl.ds(cid * PSIDE + sid * RPT, RPT)])


_deg_call = pl.kernel(
    _deg_body,
    out_type=jax.ShapeDtypeStruct((NP,), _f32),
    mesh=_mesh,
    scratch_types=[
        pltpu.VMEM_SHARED((PSIDE,), _f32),
        pltpu.VMEM((NCH, CH), jnp.int32),
        pltpu.VMEM((CH,), _f32),
        pltpu.VMEM((RPT,), _f32),
    ],
)


def _prep_body(deg_ref, emb_ref, dinv_ref, dinv2_ref, z0_ref, z1_ref):
  d = deg_ref[...]
  dinv = jnp.where(d > 0.0, lax.rsqrt(jnp.maximum(d, 1e-30)), 0.0)
  dinv_ref[...] = dinv
  dinv2_ref[...] = dinv * dinv
  z0_ref[...] = emb_ref[:, :EC] * dinv
  z1_ref[...] = emb_ref[:, EC:] * dinv


_prep_call = pl.pallas_call(
    _prep_body,
    out_shape=(
        jax.ShapeDtypeStruct((NP, 1), _f32),
        jax.ShapeDtypeStruct((NP, 1), _f32),
        jax.ShapeDtypeStruct((NP, EC), _f32),
        jax.ShapeDtypeStruct((NP, EC), _f32),
    ),
)


def _layer_body(z0, z1, rowsq, colsq, s0out, s1out,
                acc_sh, z_sh, rv, cv, *gbs):
  gbufs = gbs[:NBUF]
  gsems = gbs[NBUF:2 * NBUF]
  ssems = gbs[2 * NBUF:]
  sbuf = gbufs[0]
  cid = lax.axis_index("c")
  sid = lax.axis_index("s")
  gbase = cid * PSIDE + sid * RPT
  lbase = sid * RPT
  sbase = (1 - cid) * PSIDE + sid * RPT

  pltpu.sync_copy(rowsq.at[cid, sid], rv)
  pltpu.sync_copy(colsq.at[cid, sid], cv)

  for h, zin, so in ((0, z0, s0out), (1, z1, s1out)):
    pltpu.sync_copy(zin.at[pl.ds(sbase, RPT)], z_sh.at[pl.ds(lbase, RPT)])

    @pl.loop(0, WB)
    def _(r):
      for v in range(EC // VL):
        sbuf[r, pl.ds(v * VL, VL)] = jnp.zeros((VL,), _f32)

    for k in range(RPT // WB):
      pltpu.sync_copy(sbuf, acc_sh.at[pl.ds(lbase + k * WB, WB)])
    plsc.subcore_barrier()

    gh = [None] * NBUF
    sh = [None] * NBUF
    for j in range(NCH + LAG):
      if j < NCH:
        b = j % NBUF
        if sh[b] is not None:
          sh[b].wait()
          sh[b] = None
        gh[b] = pltpu.async_copy(z_sh.at[cv.at[j]], gbufs[b], gsems[b])
      if j >= LAG:
        jj = j - LAG
        b = jj % NBUF
        gh[b].wait()
        gh[b] = None
        sh[b] = pltpu.async_copy(gbufs[b], acc_sh.at[rv.at[jj]], ssems[b],
                                 add=True)
    for b in range(NBUF):
      if sh[b] is not None:
        sh[b].wait()

    plsc.subcore_barrier()

    pltpu.sync_copy(acc_sh.at[pl.ds(lbase, RPT)], so.at[pl.ds(gbase, RPT)])


_layer_call = pl.kernel(
    _layer_body,
    out_type=(
        jax.ShapeDtypeStruct((NP, EC), _f32),
        jax.ShapeDtypeStruct((NP, EC), _f32),
    ),
    mesh=_mesh,
    scratch_types=[
        pltpu.VMEM_SHARED((PSIDE, EC), _f32),
        pltpu.VMEM_SHARED((PSIDE, EC), _f32),
        pltpu.VMEM((NCH, CH), jnp.int32),
        pltpu.VMEM((NCH, CH), jnp.int32),
    ] + [pltpu.VMEM((CH, EC), _f32)] * NBUF
      + [pltpu.SemaphoreType.DMA] * (2 * NBUF),
)


def _scale_body(s0, s1, acc, dinv, dinv2, z0o, z1o, acco):
  di = dinv[...]
  di2 = dinv2[...]
  a = s0[...]
  b = s1[...]
  z0o[...] = di2 * a
  z1o[...] = di2 * b
  acco[...] = acc[...] + jnp.concatenate([di * a, di * b], axis=1)


_scale_call = pl.pallas_call(
    _scale_body,
    out_shape=(
        jax.ShapeDtypeStruct((NP, EC), _f32),
        jax.ShapeDtypeStruct((NP, EC), _f32),
        jax.ShapeDtypeStruct((NP, EMB), _f32),
    ),
)


def _out_body(acc, uidx, iidx, uout, iout, idxv, buf, sem):
  cid = lax.axis_index("c")
  sid = lax.axis_index("s")
  base = (cid * TPS + sid) * BPW

  for src, dst in ((uidx, uout), (iidx, iout)):
    pltpu.sync_copy(src.at[pl.ds(base, BPW)], idxv)
    pltpu.async_copy(acc.at[idxv], buf, sem).wait()

    @pl.loop(0, BPW)
    def _(r):
      for v in range(EMB // VL):
        buf[r, pl.ds(v * VL, VL)] = buf[r, pl.ds(v * VL, VL)] * (1.0 / 6.0)

    pltpu.sync_copy(buf, dst.at[pl.ds(base, BPW)])


_out_call = pl.kernel(
    _out_body,
    out_type=(
        jax.ShapeDtypeStruct((NB, EMB), _f32),
        jax.ShapeDtypeStruct((NB, EMB), _f32),
    ),
    mesh=_mesh,
    scratch_types=[
        pltpu.VMEM((BPW,), jnp.int32),
        pltpu.VMEM((BPW, EMB), _f32),
        pltpu.SemaphoreType.DMA,
    ],
)


def _pack_side(r_local, c_local):
  npad = TPS * EPT - NE
  r = jnp.concatenate([r_local, jnp.full((npad,), PSIDE - 1, jnp.int32)])
  c = jnp.concatenate([c_local, jnp.full((npad,), PSIDE - 1, jnp.int32)])
  return r.reshape(TPS, NCH, CH), c.reshape(TPS, NCH, CH)


def kernel(users, items, user_table, item_table, rows, cols, vals):
  users = users.astype(jnp.int32)
  items = items.astype(jnp.int32)
  rows = rows.astype(jnp.int32)
  cols = cols.astype(jnp.int32)

  emb = jnp.concatenate([
      jnp.pad(user_table.astype(_f32), ((0, PSIDE - NU), (0, 0))),
      jnp.pad(item_table.astype(_f32), ((0, PSIDE - NI), (0, 0))),
  ], axis=0)

  r0, c0 = _pack_side(rows[:NE], cols[:NE] - NU)
  r1, c1 = _pack_side(rows[NE:] - NU, cols[NE:])
  rowsq = jnp.stack([r0, r1])
  colsq = jnp.stack([c0, c1])

  deg = _deg_call(rowsq).reshape(NP, 1)
  dinv, dinv2, z0h, z1h = _prep_call(deg, emb)

  acc = emb
  for _ in range(NL):
    s0, s1 = _layer_call(z0h, z1h, rowsq, colsq)
    z0h, z1h, acc = _scale_call(s0, s1, acc, dinv, dinv2)

  return _out_call(acc, users, items + PSIDE)
